# baseline (speedup 1.0000x reference)
import jax
import jax.numpy as jnp
from jax import lax
from jax.experimental import pallas as pl
from jax.experimental.pallas import tpu as pltpu
from jax.experimental.pallas import tpu_sc as plsc

M, N, K = 1024, 32768, 8192
NC, NS = 2, 16
NW = NC * NS
ROWS_PER_W = M // NW
LANES = 16
WINDOWS = K // LANES
HALF = K // 2
HWIN = WINDOWS // 2
NBUF = 3


def _scatter_half(ind_v, x0_v, x1_v, row_v, roff, lo):
    @plsc.parallel_loop(lo * LANES, (lo + HWIN) * LANES, LANES, unroll=8)
    def _(o):
        packed = ind_v[pl.ds(o, LANES)]
        idx1 = (packed & 0xFFFF) + roff
        idx2 = lax.shift_right_logical(packed, 16) + roff
        plsc.addupdate_scatter(row_v, [idx1], x0_v[pl.ds(o, LANES)])
        plsc.addupdate_scatter(row_v, [idx2], x1_v[pl.ds(o, LANES)])


def _body(out_hbm, x0_hbm, x1_hbm, ind_hbm, out_o,
          ind_v, row_v, x0_v, x1_v,
          sem_row, sem_x, sem_out):
    wid = lax.axis_index("s") * NC + lax.axis_index("c")
    base = wid * ROWS_PER_W

    pltpu.sync_copy(ind_hbm, ind_v)

    def row_in(r):
        return pltpu.make_async_copy(
            out_hbm.at[base + r],
            row_v.at[pl.ds(lax.rem(r, NBUF) * N, N)], sem_row)

    def x_in(r, h):
        sl = pl.ds(h * HALF, HALF)
        return (
            pltpu.make_async_copy(
                x0_hbm.at[base + r, sl], x0_v.at[sl], sem_x),
            pltpu.make_async_copy(
                x1_hbm.at[base + r, sl], x1_v.at[sl], sem_x),
        )

    def row_out(r):
        return pltpu.make_async_copy(
            row_v.at[pl.ds(lax.rem(r, NBUF) * N, N)],
            out_o.at[base + r], sem_out)

    row_in(0).start()
    for c in x_in(0, 0) + x_in(0, 1):
        c.start()

    def row_body(r, carry):
        @pl.when(r >= 2)
        def _():
            row_out(r - 2).wait()

        @pl.when(r + 1 < ROWS_PER_W)
        def _():
            row_in(r + 1).start()

        row_in(r).wait()
        roff = lax.rem(r, NBUF) * N

        for c in x_in(r, 0):
            c.wait()
        _scatter_half(ind_v, x0_v, x1_v, row_v, roff, 0)

        @pl.when(r + 1 < ROWS_PER_W)
        def _():
            for c in x_in(r + 1, 0):
                c.start()

        for c in x_in(r, 1):
            c.wait()
        _scatter_half(ind_v, x0_v, x1_v, row_v, roff, HWIN)

        @pl.when(r + 1 < ROWS_PER_W)
        def _():
            for c in x_in(r + 1, 1):
                c.start()

        row_out(r).start()
        return carry

    lax.fori_loop(0, ROWS_PER_W, row_body, 0)

    row_out(ROWS_PER_W - 2).wait()
    row_out(ROWS_PER_W - 1).wait()


def kernel(out, x0, x1, ind1, ind2):
    mesh = plsc.VectorSubcoreMesh(
        core_axis_name="c", subcore_axis_name="s",
        num_cores=NC, num_subcores=NS)
    f = pl.kernel(
        _body,
        out_type=jax.ShapeDtypeStruct((M, N), jnp.float32),
        mesh=mesh,
        scratch_types=[
            pltpu.VMEM((K,), jnp.int32),
            pltpu.VMEM((NBUF * N,), jnp.float32),
            pltpu.VMEM((K,), jnp.float32),
            pltpu.VMEM((K,), jnp.float32),
            pltpu.SemaphoreType.DMA,
            pltpu.SemaphoreType.DMA,
            pltpu.SemaphoreType.DMA,
        ],
        compiler_params=pltpu.CompilerParams(needs_layout_passes=False),
    )
    packed = (ind1.astype(jnp.int32) & 0xFFFF) | (
        ind2.astype(jnp.int32) << 16)
    return f(out, x0, x1, packed)
